# baseline (speedup 1.0000x reference)
import functools

import jax
import jax.numpy as jnp
from jax import lax
from jax.experimental import pallas as pl
from jax.experimental.pallas import tpu as pltpu
from jax.experimental.pallas import tpu_sc as plsc

BG = 8
CH = 2048


def _sc_gather_rows(table, idx_flat):
    tot = idx_flat.shape[0]
    width = table.shape[-1]
    info = plsc.get_sparse_core_info()
    nw = info.num_cores * info.num_subcores
    b_per_w = tot // nw
    mesh = plsc.VectorSubcoreMesh(core_axis_name="c", subcore_axis_name="s")

    @functools.partial(
        pl.kernel,
        mesh=mesh,
        out_type=jax.ShapeDtypeStruct((tot, width), jnp.float32),
        scratch_types=[
            pltpu.VMEM((b_per_w,), jnp.int32),
            pltpu.VMEM((b_per_w, width), jnp.float32),
            pltpu.SemaphoreType.DMA,
        ],
    )
    def gather_k(table_hbm, idx_hbm, out_hbm, idx_v, rows_v, sem):
        wid = lax.axis_index("s") * info.num_cores + lax.axis_index("c")
        base = wid * b_per_w
        pltpu.sync_copy(idx_hbm.at[pl.ds(base, b_per_w)], idx_v)
        pltpu.async_copy(table_hbm.at[idx_v], rows_v, sem).wait()
        pltpu.sync_copy(rows_v, out_hbm.at[pl.ds(base, b_per_w)])

    return gather_k(table, idx_flat)


_CONTRACT0 = (((0,), (0,)), ((), ()))


def _encode_body(xt_ref, dem_ref, w_ref, bcol_ref, brow_ref,
                 outct_ref, pairs_ref, sums_ref, acc_ref):
    ci = pl.program_id(1)
    w = w_ref[...]
    for j in range(BG):
        xt = xt_ref[j]
        enc_t = jnp.maximum(
            lax.dot_general(w, xt, _CONTRACT0,
                            preferred_element_type=jnp.float32)
            + bcol_ref[...], 0.0)
        dem_row = dem_ref[j]
        outct_ref[:, j, :] = jnp.concatenate([enc_t, dem_row], axis=0)
        enc_nm = jnp.maximum(
            lax.dot_general(xt, w, _CONTRACT0,
                            preferred_element_type=jnp.float32)
            + brow_ref[...], 0.0)
        u32q = lax.bitcast_convert_type(
            enc_nm.astype(jnp.bfloat16), jnp.uint16).astype(jnp.uint32)
        q = enc_nm.shape[0] // 4
        p_a = (u32q[:q] << 16) | u32q[q:2 * q]
        p_b = (u32q[2 * q:3 * q] << 16) | u32q[3 * q:]
        pairs_ref[j] = lax.bitcast_convert_type(
            jnp.concatenate([p_a, p_b], axis=1), jnp.float32)
        esum = jnp.transpose(jnp.sum(enc_t, axis=1, keepdims=True))
        dsum = jnp.sum(dem_row, axis=1, keepdims=True)
        pad = jnp.zeros((1, 63), jnp.float32)
        srow = jnp.concatenate([esum, dsum, pad], axis=1)
        total = jnp.where(ci == 0, srow, acc_ref[pl.ds(j, 1)] + srow)
        acc_ref[pl.ds(j, 1)] = total
        sums_ref[j] = total


def _vehicle_body(n_nodes, g_ref, up_ref, cap_ref, t_ref, mt_ref, s_ref,
                  outvt_ref):
    g = g_ref[...]
    d = g.shape[-1] // 2
    slot = up_ref[...]
    u = lax.bitcast_convert_type(g, jnp.uint32)
    ua = jnp.where(slot < 2.0, u[:, :, :d], u[:, :, d:])
    par = slot - 2.0 * jnp.floor(slot * 0.5)
    u16v = jnp.where(par < 0.5, ua >> 16, ua & 0xFFFF)
    venc = lax.bitcast_convert_type(u16v << 16, jnp.float32)
    rows = jnp.concatenate([venc, cap_ref[...], t_ref[...]], axis=2)
    s = s_ref[...]
    s2 = s[:, 0, :]
    mean = s2[:, :d] * (1.0 / n_nodes)
    dsum = s2[:, d:d + 1]
    row0 = jnp.concatenate([mean, dsum, mt_ref[...]], axis=1)
    outvt_ref[...] = jnp.concatenate([row0[None], rows], axis=0)


def kernel(batch_node_features, batch_vehicle_positions, batch_remaining_capacities,
           batch_time_elapsed, batch_customer_max_time, batch_customer_remaining_demands,
           W, b):
    B, N, D_IN = batch_node_features.shape
    D = W.shape[1]
    V = batch_vehicle_positions.shape[1]
    N_CUST = batch_customer_remaining_demands.shape[1]
    half = N // 2

    xt = jnp.transpose(batch_node_features, (0, 2, 1))
    dem3 = jnp.pad(batch_customer_remaining_demands,
                   ((0, 0), (0, N - N_CUST))).reshape(B, 1, N)
    bcol = b[:, None]
    brow = b[None, :]

    n_steps = B // BG
    n_chunks = N // CH
    outct, pairs, sums = pl.pallas_call(
        _encode_body,
        grid=(n_steps, n_chunks),
        in_specs=[
            pl.BlockSpec((BG, D_IN, CH), lambda i, ci: (i, 0, ci)),
            pl.BlockSpec((BG, 1, CH), lambda i, ci: (i, 0, ci)),
            pl.BlockSpec((D_IN, D), lambda i, ci: (0, 0)),
            pl.BlockSpec((D_IN, 1), lambda i, ci: (0, 0)),
            pl.BlockSpec((1, D), lambda i, ci: (0, 0)),
        ],
        out_specs=[
            pl.BlockSpec((D + 1, BG, CH), lambda i, ci: (0, i, ci)),
            pl.BlockSpec((BG, CH // 4, 2 * D), lambda i, ci: (i, ci, 0)),
            pl.BlockSpec((BG, 1, 2 * D), lambda i, ci: (i, 0, 0)),
        ],
        out_shape=[
            jax.ShapeDtypeStruct((D + 1, B, N), jnp.float32),
            jax.ShapeDtypeStruct((B, N // 4, 2 * D), jnp.float32),
            jax.ShapeDtypeStruct((B, 1, 2 * D), jnp.float32),
        ],
        scratch_shapes=[pltpu.VMEM((BG, 2 * D), jnp.float32)],
    )(xt, dem3, W, bcol, brow)
    outc = jnp.transpose(outct, (1, 2, 0))

    post = jnp.transpose(batch_vehicle_positions, (1, 0)).astype(jnp.int32)
    ci_t = post // CH
    m_t = post % CH
    quarter = N // 4
    pair_row = ci_t * (CH // 4) + m_t % (CH // 4)
    pair_idx = (pair_row
                + (jnp.arange(B, dtype=jnp.int32) * quarter)[None, :]).reshape(-1)
    gathered = _sc_gather_rows(pairs.reshape(B * (N // 4), 2 * D), pair_idx)
    g3 = gathered.reshape(V, B, 2 * D)

    up3 = (m_t // (CH // 4)).astype(jnp.float32)[:, :, None]
    cap3 = jnp.transpose(batch_remaining_capacities, (1, 0))[:, :, None]
    t3 = jnp.transpose(batch_time_elapsed, (1, 0))[:, :, None]
    mt2 = batch_customer_max_time[:, None]
    outvt = pl.pallas_call(
        functools.partial(_vehicle_body, N),
        grid=(1,),
        in_specs=[
            pl.BlockSpec((V, B, 2 * D), lambda i: (0, 0, 0)),
            pl.BlockSpec((V, B, 1), lambda i: (0, 0, 0)),
            pl.BlockSpec((V, B, 1), lambda i: (0, 0, 0)),
            pl.BlockSpec((V, B, 1), lambda i: (0, 0, 0)),
            pl.BlockSpec((B, 1), lambda i: (0, 0)),
            pl.BlockSpec((B, 1, 2 * D), lambda i: (0, 0, 0)),
        ],
        out_specs=pl.BlockSpec((V + 1, B, D + 2), lambda i: (0, 0, 0)),
        out_shape=jax.ShapeDtypeStruct((V + 1, B, D + 2), jnp.float32),
    )(g3, up3, cap3, t3, mt2, sums)
    outv = jnp.transpose(outvt, (1, 0, 2))

    return (outv, outc)


# speedup vs baseline: 1.2690x; 1.2690x over previous
import functools

import jax
import jax.numpy as jnp
from jax import lax
from jax.experimental import pallas as pl
from jax.experimental.pallas import tpu as pltpu
from jax.experimental.pallas import tpu_sc as plsc

BG = 8
CH = 2048


def _sc_gather_rows(table, idx_flat):
    tot = idx_flat.shape[0]
    width = table.shape[-1]
    info = plsc.get_sparse_core_info()
    nw = info.num_cores * info.num_subcores
    b_per_w = tot // nw
    mesh = plsc.VectorSubcoreMesh(core_axis_name="c", subcore_axis_name="s")

    @functools.partial(
        pl.kernel,
        mesh=mesh,
        out_type=jax.ShapeDtypeStruct((tot, width), jnp.float32),
        scratch_types=[
            pltpu.VMEM((b_per_w,), jnp.int32),
            pltpu.VMEM((b_per_w, width), jnp.float32),
            pltpu.SemaphoreType.DMA,
        ],
    )
    def gather_k(table_hbm, idx_hbm, out_hbm, idx_v, rows_v, sem):
        wid = lax.axis_index("s") * info.num_cores + lax.axis_index("c")
        base = wid * b_per_w
        pltpu.sync_copy(idx_hbm.at[pl.ds(base, b_per_w)], idx_v)
        pltpu.async_copy(table_hbm.at[idx_v], rows_v, sem).wait()
        pltpu.sync_copy(rows_v, out_hbm.at[pl.ds(base, b_per_w)])

    return gather_k(table, idx_flat)


_CONTRACT0 = (((0,), (0,)), ((), ()))


def _encode_body(d_model, xt_ref, dem_ref, bd_ref, bcol_ref, brow_ref,
                 outct_ref, pairs_ref, sums_ref, acc_ref):
    ci = pl.program_id(1)
    bd = bd_ref[...]
    xall = xt_ref[...].reshape(BG * d_model, -1)
    ch = xall.shape[1]
    enc_t_all = jnp.maximum(
        lax.dot_general(bd, xall, _CONTRACT0,
                        preferred_element_type=jnp.float32)
        + bcol_ref[...], 0.0)
    enc_nm_all = jnp.maximum(
        lax.dot_general(xall, bd, _CONTRACT0,
                        preferred_element_type=jnp.float32)
        + brow_ref[...], 0.0)
    u32q = lax.bitcast_convert_type(
        enc_nm_all.astype(jnp.bfloat16), jnp.uint16).astype(jnp.uint32)
    q = ch // 4
    p_a = (u32q[:q] << 16) | u32q[q:2 * q]
    p_b = (u32q[2 * q:3 * q] << 16) | u32q[3 * q:]
    esum_all = jnp.transpose(
        jnp.sum(enc_t_all, axis=1, keepdims=True))
    pad = jnp.zeros((1, 63), jnp.float32)
    for j in range(BG):
        lo, hi = j * d_model, (j + 1) * d_model
        dem_row = dem_ref[j]
        outct_ref[:, j, :] = jnp.concatenate(
            [enc_t_all[lo:hi], dem_row], axis=0)
        pairs_ref[j] = lax.bitcast_convert_type(
            jnp.concatenate([p_a[:, lo:hi], p_b[:, lo:hi]], axis=1),
            jnp.float32)
        dsum = jnp.sum(dem_row, axis=1, keepdims=True)
        srow = jnp.concatenate(
            [esum_all[:, lo:hi], dsum, pad], axis=1)
        total = jnp.where(ci == 0, srow, acc_ref[pl.ds(j, 1)] + srow)
        acc_ref[pl.ds(j, 1)] = total
        sums_ref[j] = total


def _vehicle_body(n_nodes, g_ref, up_ref, cap_ref, t_ref, mt_ref, s_ref,
                  outvt_ref):
    g = g_ref[...]
    d = g.shape[-1] // 2
    slot = up_ref[...][:, :, None]
    u = lax.bitcast_convert_type(g, jnp.uint32)
    ua = jnp.where(slot < 2.0, u[:, :, :d], u[:, :, d:])
    par = slot - 2.0 * jnp.floor(slot * 0.5)
    u16v = jnp.where(par < 0.5, ua >> 16, ua & 0xFFFF)
    venc = lax.bitcast_convert_type(u16v << 16, jnp.float32)
    rows = jnp.concatenate(
        [venc, cap_ref[...][:, :, None], t_ref[...][:, :, None]], axis=2)
    s = s_ref[...]
    s2 = s[:, 0, :]
    mean = s2[:, :d] * (1.0 / n_nodes)
    dsum = s2[:, d:d + 1]
    row0 = jnp.concatenate([mean, dsum, mt_ref[...]], axis=1)
    outvt_ref[...] = jnp.concatenate([row0[None], rows], axis=0)


def kernel(batch_node_features, batch_vehicle_positions, batch_remaining_capacities,
           batch_time_elapsed, batch_customer_max_time, batch_customer_remaining_demands,
           W, b):
    B, N, D_IN = batch_node_features.shape
    D = W.shape[1]
    V = batch_vehicle_positions.shape[1]
    N_CUST = batch_customer_remaining_demands.shape[1]
    half = N // 2

    xt = jnp.transpose(batch_node_features, (0, 2, 1))
    dem3 = jnp.pad(batch_customer_remaining_demands,
                   ((0, 0), (0, N - N_CUST))).reshape(B, 1, N)
    bd = jnp.kron(jnp.eye(BG, dtype=jnp.float32), W)
    bcol = jnp.tile(b, BG)[:, None]
    brow = jnp.tile(b, BG)[None, :]

    n_steps = B // BG
    n_chunks = N // CH
    outct, pairs, sums = pl.pallas_call(
        functools.partial(_encode_body, D),
        grid=(n_steps, n_chunks),
        in_specs=[
            pl.BlockSpec((BG, D_IN, CH), lambda i, ci: (i, 0, ci)),
            pl.BlockSpec((BG, 1, CH), lambda i, ci: (i, 0, ci)),
            pl.BlockSpec((BG * D_IN, BG * D), lambda i, ci: (0, 0)),
            pl.BlockSpec((BG * D_IN, 1), lambda i, ci: (0, 0)),
            pl.BlockSpec((1, BG * D), lambda i, ci: (0, 0)),
        ],
        out_specs=[
            pl.BlockSpec((D + 1, BG, CH), lambda i, ci: (0, i, ci)),
            pl.BlockSpec((BG, CH // 4, 2 * D), lambda i, ci: (i, ci, 0)),
            pl.BlockSpec((BG, 1, 2 * D), lambda i, ci: (i, 0, 0)),
        ],
        out_shape=[
            jax.ShapeDtypeStruct((D + 1, B, N), jnp.float32),
            jax.ShapeDtypeStruct((B, N // 4, 2 * D), jnp.float32),
            jax.ShapeDtypeStruct((B, 1, 2 * D), jnp.float32),
        ],
        scratch_shapes=[pltpu.VMEM((BG, 2 * D), jnp.float32)],
    )(xt, dem3, bd, bcol, brow)
    outc = jnp.transpose(outct, (1, 2, 0))

    post = jnp.transpose(batch_vehicle_positions, (1, 0)).astype(jnp.int32)
    ci_t = post // CH
    m_t = post % CH
    quarter = N // 4
    pair_row = ci_t * (CH // 4) + m_t % (CH // 4)
    pair_idx = (pair_row
                + (jnp.arange(B, dtype=jnp.int32) * quarter)[None, :]).reshape(-1)
    gathered = _sc_gather_rows(pairs.reshape(B * (N // 4), 2 * D), pair_idx)
    g3 = gathered.reshape(V, B, 2 * D)

    up3 = (m_t // (CH // 4)).astype(jnp.float32)
    cap3 = jnp.transpose(batch_remaining_capacities, (1, 0))
    t3 = jnp.transpose(batch_time_elapsed, (1, 0))
    mt2 = batch_customer_max_time[:, None]
    outvt = pl.pallas_call(
        functools.partial(_vehicle_body, N),
        grid=(1,),
        in_specs=[
            pl.BlockSpec((V, B, 2 * D), lambda i: (0, 0, 0)),
            pl.BlockSpec((V, B), lambda i: (0, 0)),
            pl.BlockSpec((V, B), lambda i: (0, 0)),
            pl.BlockSpec((V, B), lambda i: (0, 0)),
            pl.BlockSpec((B, 1), lambda i: (0, 0)),
            pl.BlockSpec((B, 1, 2 * D), lambda i: (0, 0, 0)),
        ],
        out_specs=pl.BlockSpec((V + 1, B, D + 2), lambda i: (0, 0, 0)),
        out_shape=jax.ShapeDtypeStruct((V + 1, B, D + 2), jnp.float32),
    )(g3, up3, cap3, t3, mt2, sums)
    outv = jnp.transpose(outvt, (1, 0, 2))

    return (outv, outc)
